# Initial kernel scaffold; baseline (speedup 1.0000x reference)
import jax
import jax.numpy as jnp
from jax.experimental import pallas as pl


def kernel(x, edge_index, batch, W_lc1, b_lc1, W_lc2, b_lc2, Wc1, bc1, g1, be1, Wc2, bc2, g2, be2, Wc3, bc3, g3, be3, Wo1, bo1, Wo2, bo2, Wo3, bo3):
    raise NotImplementedError("write your pallas kernel here")


# speedup vs baseline: 2.2736x; 2.2736x over previous
import functools

import jax
import jax.numpy as jnp
from jax import lax
from jax.experimental import pallas as pl
from jax.experimental.pallas import tpu as pltpu
from jax.experimental.pallas import tpu_sc as plsc

H = 64
BLK = 400
CHUNK = 128
NSEG = 5
EPS = 1e-5


def _make_count_kernel(n_tab, ec):
    ch = ec // 32
    stripe = n_tab // 16
    mesh = plsc.VectorSubcoreMesh(core_axis_name="c", subcore_axis_name="s")

    @functools.partial(
        pl.kernel, mesh=mesh,
        compiler_params=pltpu.CompilerParams(use_tc_tiling_on_sc=False),
        out_type=jax.ShapeDtypeStruct((2, n_tab, 16), jnp.float32),
        scratch_types=[
            pltpu.VMEM((ch, CHUNK), jnp.int32),
            pltpu.VMEM((CHUNK, 16), jnp.float32),
            pltpu.VMEM_SHARED((n_tab, 16), jnp.float32),
        ],
    )
    def count_kernel(dst2d, zeros16, ones16, out, idx_v, ones_v, acc_sh):
        c = lax.axis_index("c")
        s = lax.axis_index("s")
        w = c * 16 + s
        pltpu.sync_copy(zeros16.at[pl.ds(s * stripe, stripe)],
                        acc_sh.at[pl.ds(s * stripe, stripe)])
        pltpu.sync_copy(ones16, ones_v)
        pltpu.sync_copy(dst2d.at[pl.ds(w * ch, ch)], idx_v)
        plsc.subcore_barrier()

        @pl.loop(0, ch)
        def _(k):
            pltpu.sync_copy(ones_v, acc_sh.at[idx_v.at[k]], add=True)

        plsc.subcore_barrier()
        pltpu.sync_copy(acc_sh.at[pl.ds(s * stripe, stripe)],
                        out.at[c].at[pl.ds(s * stripe, stripe)])

    return count_kernel


def _make_edge_kernel(n_tab, ec):
    ch_w = ec // 16
    seg = ch_w // NSEG
    stripe = n_tab // 16
    mesh = plsc.VectorSubcoreMesh(core_axis_name="c", subcore_axis_name="s")

    outs = (jax.ShapeDtypeStruct((n_tab, H), jnp.float32),
            jax.ShapeDtypeStruct((16, H), jnp.float32))

    @functools.partial(
        pl.kernel, mesh=mesh, out_type=outs,
        compiler_params=pltpu.CompilerParams(use_tc_tiling_on_sc=False),
        scratch_types=[
            pltpu.VMEM((seg, CHUNK), jnp.int32),
            pltpu.VMEM((seg, CHUNK), jnp.int32),
            pltpu.VMEM((CHUNK, 16), jnp.float32),
            pltpu.VMEM((CHUNK, 16), jnp.float32),
            pltpu.VMEM((CHUNK, 16), jnp.float32),
            pltpu.VMEM((1, 32), jnp.float32),
            pltpu.VMEM_SHARED((n_tab, 16), jnp.float32),
        ],
    )
    def edge_kernel(a_tab, b_tab, src2d, dst2d, zeros16, s_out, t2_out,
                    si_v, di_v, a_v, b_v, h_v, t2_v, acc_sh):
        c = lax.axis_index("c")
        s = lax.axis_index("s")
        z16 = jnp.zeros((1, 16), jnp.float32)

        for p in range(2):
            q = c * 2 + p
            pltpu.sync_copy(zeros16.at[pl.ds(s * stripe, stripe)],
                            acc_sh.at[pl.ds(s * stripe, stripe)])
            t2_v[pl.ds(0, 1), pl.ds(p * 16, 16)] = z16
            plsc.subcore_barrier()

            for g in range(NSEG):
                base = s * ch_w + g * seg
                pltpu.sync_copy(src2d.at[pl.ds(base, seg)], si_v)
                pltpu.sync_copy(dst2d.at[pl.ds(base, seg)], di_v)

                @pl.loop(0, seg)
                def _(k):
                    pltpu.sync_copy(a_tab.at[q].at[di_v.at[k]], a_v)
                    pltpu.sync_copy(b_tab.at[q].at[si_v.at[k]], b_v)

                    def row(r, t2a):
                        s0 = (pl.ds(r, 1), pl.ds(0, 16))
                        v0 = a_v[s0] + b_v[s0]
                        h0 = jnp.where(v0 > 0, v0, jnp.exp(v0) - 1.0)
                        h_v[s0] = h0
                        return t2a + h0 * h0

                    t2a = lax.fori_loop(0, CHUNK, row, z16)
                    t2_v[pl.ds(0, 1), pl.ds(p * 16, 16)] = \
                        t2_v[pl.ds(0, 1), pl.ds(p * 16, 16)] + t2a
                    pltpu.sync_copy(h_v, acc_sh.at[di_v.at[k]], add=True)

            plsc.subcore_barrier()
            pltpu.sync_copy(acc_sh.at[pl.ds(s * stripe, stripe)],
                            s_out.at[pl.ds(s * stripe, stripe),
                                     pl.ds(q * 16, 16)])

        pltpu.sync_copy(t2_v, t2_out.at[pl.ds(s, 1), pl.ds(c * 32, 32)])

    return edge_kernel


def _elu(t):
    return jnp.where(t > 0, t, jnp.exp(t) - 1.0)


def _enc_body(nreal, xp_ref, w1_ref, b1_ref, w2_ref, b2_ref,
              wca_ref, bca_ref, wcb_ref, xe_ref, a_ref, b_ref):
    i = pl.program_id(0)
    rows = lax.broadcasted_iota(jnp.int32, (BLK, 1), 0) + i * BLK
    mask = rows < nreal
    t = _elu(xp_ref[...] @ w1_ref[...] + b1_ref[...])
    t = _elu(t @ w2_ref[...] + b2_ref[...])
    xe = jnp.where(mask, t, 0.0)
    xe_ref[...] = xe
    a = jnp.where(mask, xe @ wca_ref[...] + bca_ref[...], 0.0)
    bb = xe @ wcb_ref[...]
    for j in range(4):
        a_ref[j] = a[:, 16 * j:16 * j + 16]
        b_ref[j] = bb[:, 16 * j:16 * j + 16]


def _f1_body(e_real, s_ref, t2_ref, st_ref, acc_ref):
    i = pl.program_id(0)

    @pl.when(i == 0)
    def _():
        acc_ref[...] = jnp.zeros_like(acc_ref)

    acc_ref[...] += jnp.sum(s_ref[...], axis=0, keepdims=True)

    @pl.when(i == pl.num_programs(0) - 1)
    def _():
        mu = acc_ref[...] / e_real
        t2 = jnp.sum(t2_ref[...], axis=0, keepdims=True)
        var = t2 / e_real - mu * mu
        invsig = lax.rsqrt(var + EPS)
        st_ref[...] = jnp.concatenate([mu, invsig], axis=0)


def _finalize(s_ref, cnt_ref, st_ref, g_ref, be_ref):
    cnt = cnt_ref[0][:, 0:1] + cnt_ref[1][:, 0:1]
    mean = s_ref[...] / jnp.maximum(cnt, 1.0)
    f = (mean - st_ref[0:1, :]) * st_ref[1:2, :] * g_ref[...] + be_ref[...]
    return jnp.where(cnt > 0, f, 0.0)


def _f2_body(nreal, s_ref, cnt_ref, st_ref, g_ref, be_ref, r_ref,
             wca_ref, bca_ref, wcb_ref, f_ref, a_ref, b_ref):
    i = pl.program_id(0)
    rows = lax.broadcasted_iota(jnp.int32, (BLK, 1), 0) + i * BLK
    mask = rows < nreal
    f = _finalize(s_ref, cnt_ref, st_ref, g_ref, be_ref)
    f_ref[...] = f
    z = f + r_ref[...]
    a = jnp.where(mask, z @ wca_ref[...] + bca_ref[...], 0.0)
    bb = jnp.where(mask, z @ wcb_ref[...], 0.0)
    for j in range(4):
        a_ref[j] = a[:, 16 * j:16 * j + 16]
        b_ref[j] = bb[:, 16 * j:16 * j + 16]


def _f3_body(s_ref, cnt_ref, st_ref, g_ref, be_ref,
             wo1_ref, bo1_ref, wo2_ref, bo2_ref, wo3_ref, bo3_ref, o_ref):
    f = _finalize(s_ref, cnt_ref, st_ref, g_ref, be_ref)
    h = _elu(f @ wo1_ref[...] + bo1_ref[...])
    h = _elu(h @ wo2_ref[...] + bo2_ref[...])
    o_ref[...] = h @ wo3_ref[...] + bo3_ref[...]


def _full(shape):
    nd = len(shape)
    return pl.BlockSpec(shape, lambda i: (0,) * nd)


def kernel(x, edge_index, batch, W_lc1, b_lc1, W_lc2, b_lc2,
           Wc1, bc1, g1, be1, Wc2, bc2, g2, be2, Wc3, bc3, g3, be3,
           Wo1, bo1, Wo2, bo2, Wo3, bo3):
    n = x.shape[0]
    e = edge_index.shape[1]
    n_tab = ((n + 1 + 3199) // 3200) * 3200
    e_pad = ((e + 256 * CHUNK - 1) // (256 * CHUNK)) * (256 * CHUNK)
    ec = e_pad // CHUNK
    grid = n_tab // BLK

    f32 = jnp.float32

    xp = jnp.pad(x, ((0, n_tab - n), (0, 1)))
    w1p = jnp.pad(W_lc1, ((0, 1), (0, 0)))
    pad_idx = jnp.full((e_pad - e,), n, jnp.int32)
    src2d = jnp.concatenate([edge_index[0], pad_idx]).reshape(ec, CHUNK)
    dst2d = jnp.concatenate([edge_index[1], pad_idx]).reshape(ec, CHUNK)
    zeros16 = jnp.zeros((n_tab, 16), f32)
    ones16 = jnp.zeros((CHUNK, 16), f32).at[:, 0].set(1.0)
    row = lambda v: v.reshape(1, -1)
    wca = [Wc1[:H] - Wc1[H:], Wc2[:H] - Wc2[H:], Wc3[:H] - Wc3[H:]]
    wcb = [Wc1[H:], Wc2[H:], Wc3[H:]]
    bca = [row(bc1), row(bc2), row(bc3)]
    gs = [row(g1), row(g2), row(g3)]
    bes = [row(be1), row(be2), row(be3)]

    count_kernel = _make_count_kernel(n_tab, ec)
    edge_kernel = _make_edge_kernel(n_tab, ec)

    st_tab = jax.ShapeDtypeStruct((4, n_tab, 16), f32)
    st_blk = pl.BlockSpec((4, BLK, 16), lambda i: (0, i, 0))

    xe, a1, b1 = pl.pallas_call(
        functools.partial(_enc_body, n),
        grid=(grid,),
        in_specs=[pl.BlockSpec((BLK, 16), lambda i: (i, 0)),
                  _full((16, H)), _full((1, H)), _full((H, H)), _full((1, H)),
                  _full((H, H)), _full((1, H)), _full((H, H))],
        out_specs=[pl.BlockSpec((BLK, H), lambda i: (i, 0)), st_blk, st_blk],
        out_shape=[jax.ShapeDtypeStruct((n_tab, H), f32), st_tab, st_tab],
    )(xp, w1p, row(b_lc1), W_lc2, row(b_lc2), wca[0], bca[0], wcb[0])

    cntp = count_kernel(dst2d, zeros16, ones16)

    def run_layer(a_tab, b_tab):
        s_sum, t2p = edge_kernel(a_tab, b_tab, src2d, dst2d, zeros16)
        st = pl.pallas_call(
            functools.partial(_f1_body, float(e)),
            grid=(grid,),
            in_specs=[pl.BlockSpec((BLK, H), lambda i: (i, 0)),
                      _full((16, H))],
            out_specs=_full((2, H)),
            out_shape=jax.ShapeDtypeStruct((2, H), f32),
            scratch_shapes=[pltpu.VMEM((1, H), f32)],
        )(s_sum, t2p)
        return s_sum, st

    cnt_blk = pl.BlockSpec((2, BLK, 16), lambda i: (0, i, 0))
    s_blk = pl.BlockSpec((BLK, H), lambda i: (i, 0))

    s1, st1 = run_layer(a1, b1)
    f1, a2, b2 = pl.pallas_call(
        functools.partial(_f2_body, n),
        grid=(grid,),
        in_specs=[s_blk, cnt_blk, _full((2, H)), _full((1, H)),
                  _full((1, H)), pl.BlockSpec((BLK, H), lambda i: (i, 0)),
                  _full((H, H)), _full((1, H)), _full((H, H))],
        out_specs=[pl.BlockSpec((BLK, H), lambda i: (i, 0)), st_blk, st_blk],
        out_shape=[jax.ShapeDtypeStruct((n_tab, H), f32), st_tab, st_tab],
    )(s1, cntp, st1, gs[0], bes[0], xe, wca[1], bca[1], wcb[1])

    s2, st2 = run_layer(a2, b2)
    f2, a3, b3 = pl.pallas_call(
        functools.partial(_f2_body, n),
        grid=(grid,),
        in_specs=[s_blk, cnt_blk, _full((2, H)), _full((1, H)),
                  _full((1, H)), pl.BlockSpec((BLK, H), lambda i: (i, 0)),
                  _full((H, H)), _full((1, H)), _full((H, H))],
        out_specs=[pl.BlockSpec((BLK, H), lambda i: (i, 0)), st_blk, st_blk],
        out_shape=[jax.ShapeDtypeStruct((n_tab, H), f32), st_tab, st_tab],
    )(s2, cntp, st2, gs[1], bes[1], f1, wca[2], bca[2], wcb[2])

    s3, st3 = run_layer(a3, b3)
    o_full = pl.pallas_call(
        _f3_body,
        grid=(grid,),
        in_specs=[s_blk, cnt_blk, _full((2, H)), _full((1, H)),
                  _full((1, H)),
                  _full((H, 32)), _full((1, 32)), _full((32, 16)),
                  _full((1, 16)), _full((16, 8)), _full((1, 8))],
        out_specs=pl.BlockSpec((BLK, 8), lambda i: (i, 0)),
        out_shape=jax.ShapeDtypeStruct((n_tab, 8), f32),
    )(s3, cntp, st3, gs[2], bes[2],
      Wo1, row(bo1), Wo2, row(bo2), Wo3, row(bo3))

    return (o_full[:n], batch)


# speedup vs baseline: 4.7485x; 2.0886x over previous
import functools

import jax
import jax.numpy as jnp
from jax import lax
from jax.experimental import pallas as pl
from jax.experimental.pallas import tpu as pltpu
from jax.experimental.pallas import tpu_sc as plsc

H = 64
BLK = 400
CHUNK = 128
NSEG = 5
EPS = 1e-5


def _make_count_kernel(n_tab, ec):
    ch = ec // 32
    stripe = n_tab // 16
    mesh = plsc.VectorSubcoreMesh(core_axis_name="c", subcore_axis_name="s")

    @functools.partial(
        pl.kernel, mesh=mesh,
        compiler_params=pltpu.CompilerParams(use_tc_tiling_on_sc=False),
        out_type=jax.ShapeDtypeStruct((2, n_tab, 16), jnp.float32),
        scratch_types=[
            pltpu.VMEM((ch, CHUNK), jnp.int32),
            pltpu.VMEM((CHUNK, 16), jnp.float32),
            pltpu.VMEM_SHARED((n_tab, 16), jnp.float32),
        ],
    )
    def count_kernel(dst2d, zeros16, ones16, out, idx_v, ones_v, acc_sh):
        c = lax.axis_index("c")
        s = lax.axis_index("s")
        w = c * 16 + s
        pltpu.sync_copy(zeros16.at[pl.ds(s * stripe, stripe)],
                        acc_sh.at[pl.ds(s * stripe, stripe)])
        pltpu.sync_copy(ones16, ones_v)
        pltpu.sync_copy(dst2d.at[pl.ds(w * ch, ch)], idx_v)
        plsc.subcore_barrier()

        @pl.loop(0, ch)
        def _(k):
            pltpu.sync_copy(ones_v, acc_sh.at[idx_v.at[k]], add=True)

        plsc.subcore_barrier()
        pltpu.sync_copy(acc_sh.at[pl.ds(s * stripe, stripe)],
                        out.at[c].at[pl.ds(s * stripe, stripe)])

    return count_kernel


def _make_edge_kernel(n_tab, ec):
    ch_w = ec // 16
    seg = ch_w // NSEG
    stripe = n_tab // 16
    mesh = plsc.VectorSubcoreMesh(core_axis_name="c", subcore_axis_name="s")

    outs = (jax.ShapeDtypeStruct((n_tab, H), jnp.float32),
            jax.ShapeDtypeStruct((16, H), jnp.float32))

    @functools.partial(
        pl.kernel, mesh=mesh, out_type=outs,
        compiler_params=pltpu.CompilerParams(use_tc_tiling_on_sc=False),
        scratch_types=[
            pltpu.VMEM((seg, CHUNK), jnp.int32),
            pltpu.VMEM((seg, CHUNK), jnp.int32),
            pltpu.VMEM((CHUNK, 16), jnp.float32),
            pltpu.VMEM((CHUNK, 16), jnp.float32),
            pltpu.VMEM((CHUNK, 16), jnp.float32),
            pltpu.VMEM((CHUNK, 16), jnp.float32),
            pltpu.VMEM((CHUNK, 16), jnp.float32),
            pltpu.VMEM((CHUNK, 16), jnp.float32),
            pltpu.VMEM((1, 32), jnp.float32),
            pltpu.VMEM_SHARED((n_tab, 16), jnp.float32),
            pltpu.SemaphoreType.DMA,
            pltpu.SemaphoreType.DMA,
            pltpu.SemaphoreType.DMA,
            pltpu.SemaphoreType.DMA,
        ],
    )
    def edge_kernel(a_tab, b_tab, src2d, dst2d, zeros16, s_out, t2_out,
                    si_v, di_v, a0_v, a1_v, b0_v, b1_v, h0_v, h1_v,
                    t2_v, acc_sh, gsem0, gsem1, ssem0, ssem1):
        c = lax.axis_index("c")
        s = lax.axis_index("s")
        z16 = jnp.zeros((1, 16), jnp.float32)
        slots = ((a0_v, b0_v, h0_v, gsem0, ssem0),
                 (a1_v, b1_v, h1_v, gsem1, ssem1))

        for p in range(2):
            q = c * 2 + p
            a_q = a_tab.at[q]
            b_q = b_tab.at[q]
            pltpu.sync_copy(zeros16.at[pl.ds(s * stripe, stripe)],
                            acc_sh.at[pl.ds(s * stripe, stripe)])
            t2_v[pl.ds(0, 1), pl.ds(p * 16, 16)] = z16
            plsc.subcore_barrier()

            for g in range(NSEG):
                base = s * ch_w + g * seg
                pltpu.sync_copy(src2d.at[pl.ds(base, seg)], si_v)
                pltpu.sync_copy(dst2d.at[pl.ds(base, seg)], di_v)

                for b, (a_v, b_v, _, gsem, _) in enumerate(slots):
                    pltpu.async_copy(a_q.at[di_v.at[b]], a_v, gsem)
                    pltpu.async_copy(b_q.at[si_v.at[b]], b_v, gsem)

                @pl.loop(0, seg, step=2)
                def _(j):
                    for b, (a_v, b_v, h_v, gsem, ssem) in enumerate(slots):
                        k = j + b
                        pltpu.make_async_copy(a_q.at[di_v.at[k]], a_v,
                                              gsem).wait()
                        pltpu.make_async_copy(b_q.at[si_v.at[k]], b_v,
                                              gsem).wait()

                        @pl.when(k >= 2)
                        def _():
                            pltpu.make_async_copy(
                                h_v, acc_sh.at[di_v.at[k]], ssem).wait()

                        def row(r, t2a):
                            s0 = (pl.ds(r, 1), pl.ds(0, 16))
                            v0 = a_v[s0] + b_v[s0]
                            h0 = jnp.where(v0 > 0, v0, jnp.exp(v0) - 1.0)
                            h_v[s0] = h0
                            return t2a + h0 * h0

                        t2a = lax.fori_loop(0, CHUNK, row, z16)
                        t2_v[pl.ds(0, 1), pl.ds(p * 16, 16)] = \
                            t2_v[pl.ds(0, 1), pl.ds(p * 16, 16)] + t2a
                        pltpu.async_copy(h_v, acc_sh.at[di_v.at[k]], ssem,
                                         add=True)
                        kn = jnp.minimum(k + 2, seg - 1)
                        pltpu.async_copy(a_q.at[di_v.at[kn]], a_v, gsem)
                        pltpu.async_copy(b_q.at[si_v.at[kn]], b_v, gsem)

                for b, (a_v, b_v, h_v, gsem, ssem) in enumerate(slots):
                    pltpu.make_async_copy(a_q.at[di_v.at[b]], a_v,
                                          gsem).wait()
                    pltpu.make_async_copy(b_q.at[si_v.at[b]], b_v,
                                          gsem).wait()
                    pltpu.make_async_copy(h_v, acc_sh.at[di_v.at[b]],
                                          ssem).wait()

            plsc.subcore_barrier()
            pltpu.sync_copy(acc_sh.at[pl.ds(s * stripe, stripe)],
                            s_out.at[pl.ds(s * stripe, stripe),
                                     pl.ds(q * 16, 16)])

        pltpu.sync_copy(t2_v, t2_out.at[pl.ds(s, 1), pl.ds(c * 32, 32)])

    return edge_kernel


def _elu(t):
    return jnp.where(t > 0, t, jnp.exp(t) - 1.0)


def _enc_body(nreal, xp_ref, w1_ref, b1_ref, w2_ref, b2_ref,
              wca_ref, bca_ref, wcb_ref, xe_ref, a_ref, b_ref):
    i = pl.program_id(0)
    rows = lax.broadcasted_iota(jnp.int32, (BLK, 1), 0) + i * BLK
    mask = rows < nreal
    t = _elu(xp_ref[...] @ w1_ref[...] + b1_ref[...])
    t = _elu(t @ w2_ref[...] + b2_ref[...])
    xe = jnp.where(mask, t, 0.0)
    xe_ref[...] = xe
    a = jnp.where(mask, xe @ wca_ref[...] + bca_ref[...], 0.0)
    bb = xe @ wcb_ref[...]
    for j in range(4):
        a_ref[j] = a[:, 16 * j:16 * j + 16]
        b_ref[j] = bb[:, 16 * j:16 * j + 16]


def _f1_body(e_real, s_ref, t2_ref, st_ref, acc_ref):
    i = pl.program_id(0)

    @pl.when(i == 0)
    def _():
        acc_ref[...] = jnp.zeros_like(acc_ref)

    acc_ref[...] += jnp.sum(s_ref[...], axis=0, keepdims=True)

    @pl.when(i == pl.num_programs(0) - 1)
    def _():
        mu = acc_ref[...] / e_real
        t2 = jnp.sum(t2_ref[...], axis=0, keepdims=True)
        var = t2 / e_real - mu * mu
        invsig = lax.rsqrt(var + EPS)
        st_ref[...] = jnp.concatenate([mu, invsig], axis=0)


def _finalize(s_ref, cnt_ref, st_ref, g_ref, be_ref):
    cnt = cnt_ref[0][:, 0:1] + cnt_ref[1][:, 0:1]
    mean = s_ref[...] / jnp.maximum(cnt, 1.0)
    f = (mean - st_ref[0:1, :]) * st_ref[1:2, :] * g_ref[...] + be_ref[...]
    return jnp.where(cnt > 0, f, 0.0)


def _f2_body(nreal, s_ref, cnt_ref, st_ref, g_ref, be_ref, r_ref,
             wca_ref, bca_ref, wcb_ref, f_ref, a_ref, b_ref):
    i = pl.program_id(0)
    rows = lax.broadcasted_iota(jnp.int32, (BLK, 1), 0) + i * BLK
    mask = rows < nreal
    f = _finalize(s_ref, cnt_ref, st_ref, g_ref, be_ref)
    f_ref[...] = f
    z = f + r_ref[...]
    a = jnp.where(mask, z @ wca_ref[...] + bca_ref[...], 0.0)
    bb = jnp.where(mask, z @ wcb_ref[...], 0.0)
    for j in range(4):
        a_ref[j] = a[:, 16 * j:16 * j + 16]
        b_ref[j] = bb[:, 16 * j:16 * j + 16]


def _f3_body(s_ref, cnt_ref, st_ref, g_ref, be_ref,
             wo1_ref, bo1_ref, wo2_ref, bo2_ref, wo3_ref, bo3_ref, o_ref):
    f = _finalize(s_ref, cnt_ref, st_ref, g_ref, be_ref)
    h = _elu(f @ wo1_ref[...] + bo1_ref[...])
    h = _elu(h @ wo2_ref[...] + bo2_ref[...])
    o_ref[...] = h @ wo3_ref[...] + bo3_ref[...]


def _full(shape):
    nd = len(shape)
    return pl.BlockSpec(shape, lambda i: (0,) * nd)


def kernel(x, edge_index, batch, W_lc1, b_lc1, W_lc2, b_lc2,
           Wc1, bc1, g1, be1, Wc2, bc2, g2, be2, Wc3, bc3, g3, be3,
           Wo1, bo1, Wo2, bo2, Wo3, bo3):
    n = x.shape[0]
    e = edge_index.shape[1]
    n_tab = ((n + 1 + 3199) // 3200) * 3200
    e_pad = ((e + 256 * CHUNK - 1) // (256 * CHUNK)) * (256 * CHUNK)
    ec = e_pad // CHUNK
    grid = n_tab // BLK

    f32 = jnp.float32

    xp = jnp.pad(x, ((0, n_tab - n), (0, 1)))
    w1p = jnp.pad(W_lc1, ((0, 1), (0, 0)))
    pad_idx = jnp.full((e_pad - e,), n, jnp.int32)
    src2d = jnp.concatenate([edge_index[0], pad_idx]).reshape(ec, CHUNK)
    dst2d = jnp.concatenate([edge_index[1], pad_idx]).reshape(ec, CHUNK)
    zeros16 = jnp.zeros((n_tab, 16), f32)
    ones16 = jnp.zeros((CHUNK, 16), f32).at[:, 0].set(1.0)
    row = lambda v: v.reshape(1, -1)
    wca = [Wc1[:H] - Wc1[H:], Wc2[:H] - Wc2[H:], Wc3[:H] - Wc3[H:]]
    wcb = [Wc1[H:], Wc2[H:], Wc3[H:]]
    bca = [row(bc1), row(bc2), row(bc3)]
    gs = [row(g1), row(g2), row(g3)]
    bes = [row(be1), row(be2), row(be3)]

    count_kernel = _make_count_kernel(n_tab, ec)
    edge_kernel = _make_edge_kernel(n_tab, ec)

    st_tab = jax.ShapeDtypeStruct((4, n_tab, 16), f32)
    st_blk = pl.BlockSpec((4, BLK, 16), lambda i: (0, i, 0))

    xe, a1, b1 = pl.pallas_call(
        functools.partial(_enc_body, n),
        grid=(grid,),
        in_specs=[pl.BlockSpec((BLK, 16), lambda i: (i, 0)),
                  _full((16, H)), _full((1, H)), _full((H, H)), _full((1, H)),
                  _full((H, H)), _full((1, H)), _full((H, H))],
        out_specs=[pl.BlockSpec((BLK, H), lambda i: (i, 0)), st_blk, st_blk],
        out_shape=[jax.ShapeDtypeStruct((n_tab, H), f32), st_tab, st_tab],
    )(xp, w1p, row(b_lc1), W_lc2, row(b_lc2), wca[0], bca[0], wcb[0])

    cntp = count_kernel(dst2d, zeros16, ones16)

    def run_layer(a_tab, b_tab):
        s_sum, t2p = edge_kernel(a_tab, b_tab, src2d, dst2d, zeros16)
        st = pl.pallas_call(
            functools.partial(_f1_body, float(e)),
            grid=(grid,),
            in_specs=[pl.BlockSpec((BLK, H), lambda i: (i, 0)),
                      _full((16, H))],
            out_specs=_full((2, H)),
            out_shape=jax.ShapeDtypeStruct((2, H), f32),
            scratch_shapes=[pltpu.VMEM((1, H), f32)],
        )(s_sum, t2p)
        return s_sum, st

    cnt_blk = pl.BlockSpec((2, BLK, 16), lambda i: (0, i, 0))
    s_blk = pl.BlockSpec((BLK, H), lambda i: (i, 0))

    s1, st1 = run_layer(a1, b1)
    f1, a2, b2 = pl.pallas_call(
        functools.partial(_f2_body, n),
        grid=(grid,),
        in_specs=[s_blk, cnt_blk, _full((2, H)), _full((1, H)),
                  _full((1, H)), pl.BlockSpec((BLK, H), lambda i: (i, 0)),
                  _full((H, H)), _full((1, H)), _full((H, H))],
        out_specs=[pl.BlockSpec((BLK, H), lambda i: (i, 0)), st_blk, st_blk],
        out_shape=[jax.ShapeDtypeStruct((n_tab, H), f32), st_tab, st_tab],
    )(s1, cntp, st1, gs[0], bes[0], xe, wca[1], bca[1], wcb[1])

    s2, st2 = run_layer(a2, b2)
    f2, a3, b3 = pl.pallas_call(
        functools.partial(_f2_body, n),
        grid=(grid,),
        in_specs=[s_blk, cnt_blk, _full((2, H)), _full((1, H)),
                  _full((1, H)), pl.BlockSpec((BLK, H), lambda i: (i, 0)),
                  _full((H, H)), _full((1, H)), _full((H, H))],
        out_specs=[pl.BlockSpec((BLK, H), lambda i: (i, 0)), st_blk, st_blk],
        out_shape=[jax.ShapeDtypeStruct((n_tab, H), f32), st_tab, st_tab],
    )(s2, cntp, st2, gs[1], bes[1], f1, wca[2], bca[2], wcb[2])

    s3, st3 = run_layer(a3, b3)
    o_full = pl.pallas_call(
        _f3_body,
        grid=(grid,),
        in_specs=[s_blk, cnt_blk, _full((2, H)), _full((1, H)),
                  _full((1, H)),
                  _full((H, 32)), _full((1, 32)), _full((32, 16)),
                  _full((1, 16)), _full((16, 8)), _full((1, 8))],
        out_specs=pl.BlockSpec((BLK, 8), lambda i: (i, 0)),
        out_shape=jax.ShapeDtypeStruct((n_tab, 8), f32),
    )(s3, cntp, st3, gs[2], bes[2],
      Wo1, row(bo1), Wo2, row(bo2), Wo3, row(bo3))

    return (o_full[:n], batch)
